# baseline (speedup 1.0000x reference)
import jax
import jax.numpy as jnp
from jax import lax
from jax.experimental import pallas as pl
from jax.experimental.pallas import tpu as pltpu
from jax.experimental.pallas import tpu_sc as plsc

VOCAB = 1_000_000
D = 64
BATCH = 4096
SEQ = 50

_info = plsc.get_sparse_core_info()
NC = _info.num_cores
NS = _info.num_subcores
NW = NC * NS
CH = BATCH // NW
NB = 5


def _make_lookup():
  mesh = plsc.VectorSubcoreMesh(core_axis_name="c", subcore_axis_name="s")

  @pl.kernel(
      out_type=jax.ShapeDtypeStruct((BATCH, SEQ, D), jnp.float32),
      mesh=mesh,
      compiler_params=pltpu.CompilerParams(use_tc_tiling_on_sc=False),
      scratch_types=(
          [pltpu.VMEM((SEQ, CH), jnp.int32)]
          + [pltpu.VMEM((CH, D), jnp.float32) for _ in range(NB)]
          + [pltpu.SemaphoreType.DMA for _ in range(2 * NB)]
      ),
  )
  def lookup(table_hbm, xt_hbm, out_hbm, idx_v, *bufs_sems):
    bufs = bufs_sems[:NB]
    sg = bufs_sems[NB:2 * NB]
    sw = bufs_sems[2 * NB:3 * NB]
    wid = lax.axis_index("s") * NC + lax.axis_index("c")
    b0 = wid * CH
    pltpu.sync_copy(xt_hbm.at[:, pl.ds(b0, CH)], idx_v)

    def out_slice(s):
      return out_hbm.at[pl.ds(b0, CH), s, :]

    for b in range(NB):
      pltpu.async_copy(table_hbm.at[idx_v.at[b]], bufs[b], sg[b])

    @pl.loop(0, SEQ, step=NB)
    def _chunks(s0):
      for b in range(NB):
        s = s0 + b
        pltpu.make_async_copy(table_hbm.at[idx_v.at[s]], bufs[b], sg[b]).wait()
        pltpu.async_copy(bufs[b], out_slice(s), sw[b])

        @pl.when(s + NB < SEQ)
        def _():
          pltpu.make_async_copy(bufs[b], out_slice(s), sw[b]).wait()
          pltpu.async_copy(table_hbm.at[idx_v.at[s + NB]], bufs[b], sg[b])

    for b in range(NB):
      s = SEQ - NB + b
      pltpu.make_async_copy(bufs[b], out_slice(s), sw[b]).wait()

  return lookup


_lookup = _make_lookup()


@jax.jit
def kernel(x, table):
  return _lookup(table, x.T.astype(jnp.int32))
